# baseline (speedup 1.0000x reference)
import functools

import jax
import jax.numpy as jnp
from jax import lax
from jax.experimental import pallas as pl
from jax.experimental.pallas import tpu as pltpu
from jax.experimental.pallas import tpu_sc as plsc

_L = 16
_SUP = 128
_CH = 256


def _gather_kernel(B, D, b_per_w, NC):
    mesh = plsc.VectorSubcoreMesh(core_axis_name="c", subcore_axis_name="s")
    rows_per_sup = _SUP // D
    shift = rows_per_sup.bit_length() - 1
    n_ch = b_per_w // _CH
    n_grp = _CH // _L

    @functools.partial(
        pl.kernel,
        mesh=mesh,
        out_type=jax.ShapeDtypeStruct((B, D), jnp.float32),
        compiler_params=pltpu.CompilerParams(needs_layout_passes=False),
        scratch_types=[
            pltpu.VMEM((b_per_w,), jnp.int32),
            pltpu.VMEM((_CH,), jnp.int32),
            pltpu.VMEM((_CH, _SUP), jnp.float32),
            pltpu.VMEM((b_per_w, D), jnp.float32),
            pltpu.SemaphoreType.DMA,
        ],
    )
    def k(idx_hbm, table4_hbm, out_hbm, idx_v, sup_v, rows_v, out_v, sem):
        wid = lax.axis_index("s") * NC + lax.axis_index("c")
        base = wid * b_per_w
        pltpu.sync_copy(idx_hbm.at[pl.ds(base, b_per_w)], idx_v)
        iota = lax.iota(jnp.int32, _L)

        for ch in range(n_ch):
            c0 = ch * _CH

            def sup_body(j, _):
                s = pl.multiple_of(j * _L, _L)
                sup_v[pl.ds(s, _L)] = lax.shift_right_logical(
                    idx_v[pl.ds(c0 + s, _L)], shift
                )
                return 0

            lax.fori_loop(0, n_grp, sup_body, 0)
            pltpu.async_copy(table4_hbm.at[sup_v], rows_v, sem).wait()

            def ext_body(j, _):
                s = pl.multiple_of(j * _L, _L)
                idx16 = idx_v[pl.ds(c0 + s, _L)]
                row16 = s + iota
                col_base = (idx16 & (rows_per_sup - 1)) * D
                for c in range(D):
                    val = plsc.load_gather(rows_v, [row16, col_base + c])
                    plsc.store_scatter(
                        out_v, [c0 + row16, iota * 0 + c], val
                    )
                return 0

            lax.fori_loop(0, n_grp, ext_body, 0)

        pltpu.sync_copy(out_v, out_hbm.at[pl.ds(base, b_per_w)])

    return k


def kernel(node_id, node_table):
    (B,) = node_id.shape
    V, D = node_table.shape
    info = plsc.get_sparse_core_info()
    NC, NS = info.num_cores, info.num_subcores
    NW = NC * NS
    b_per_w = B // NW
    idx = node_id.astype(jnp.int32)
    table4 = node_table.reshape(V * D // _SUP, _SUP)
    return _gather_kernel(B, D, b_per_w, NC)(idx, table4)


# speedup vs baseline: 1.0238x; 1.0238x over previous
import functools

import jax
import jax.numpy as jnp
from jax import lax
from jax.experimental import pallas as pl
from jax.experimental.pallas import tpu as pltpu
from jax.experimental.pallas import tpu_sc as plsc

_L = 16
_SUP = 128
_CH = 256


def _gather_kernel(B, D, b_per_w, NC):
    mesh = plsc.VectorSubcoreMesh(core_axis_name="c", subcore_axis_name="s")
    rows_per_sup = _SUP // D
    shift = rows_per_sup.bit_length() - 1
    n_ch = b_per_w // _CH
    n_grp = _CH // _L

    @functools.partial(
        pl.kernel,
        mesh=mesh,
        out_type=jax.ShapeDtypeStruct((D, B), jnp.float32),
        compiler_params=pltpu.CompilerParams(needs_layout_passes=False),
        scratch_types=[
            pltpu.VMEM((b_per_w,), jnp.int32),
            pltpu.VMEM((_CH,), jnp.int32),
            pltpu.VMEM((_CH, _SUP + 1), jnp.float32),
            pltpu.VMEM((D, b_per_w), jnp.float32),
            pltpu.SemaphoreType.DMA,
        ],
    )
    def k(idx_hbm, table4_hbm, out_hbm, idx_v, sup_v, rows_v, out_v, sem):
        wid = lax.axis_index("s") * NC + lax.axis_index("c")
        base = wid * b_per_w
        pltpu.sync_copy(idx_hbm.at[pl.ds(base, b_per_w)], idx_v)
        iota = lax.iota(jnp.int32, _L)

        def ch_body(ch, _):
            off = ch * _CH

            def sup_body(j, _):
                s = pl.multiple_of(j * _L, _L)
                sup_v[pl.ds(s, _L)] = lax.shift_right_logical(
                    idx_v[pl.ds(off + s, _L)], shift
                )
                return 0

            lax.fori_loop(0, n_grp, sup_body, 0)
            pltpu.async_copy(
                table4_hbm.at[sup_v], rows_v.at[:, pl.ds(0, _SUP)], sem
            ).wait()

            def ext_body(g, _):
                s = pl.multiple_of(g * _L, _L)
                idx16 = idx_v[pl.ds(off + s, _L)]
                j16 = s + iota
                col_base = (idx16 & (rows_per_sup - 1)) * D
                for f in range(D):
                    val = plsc.load_gather(rows_v, [j16, col_base + f])
                    out_v[f, pl.ds(off + s, _L)] = val
                return 0

            lax.fori_loop(0, n_grp, ext_body, 0)
            return 0

        lax.fori_loop(0, n_ch, ch_body, 0)
        pltpu.sync_copy(out_v, out_hbm.at[:, pl.ds(base, b_per_w)])

    return k


def kernel(node_id, node_table):
    (B,) = node_id.shape
    V, D = node_table.shape
    info = plsc.get_sparse_core_info()
    NC, NS = info.num_cores, info.num_subcores
    NW = NC * NS
    b_per_w = B // NW
    idx = node_id.astype(jnp.int32)
    table4 = node_table.reshape(V * D // _SUP, _SUP)
    out_t = _gather_kernel(B, D, b_per_w, NC)(idx, table4)
    return out_t.T


# speedup vs baseline: 1.6027x; 1.5654x over previous
import functools

import jax
import jax.numpy as jnp
from jax import lax
from jax.experimental import pallas as pl
from jax.experimental.pallas import tpu as pltpu
from jax.experimental.pallas import tpu_sc as plsc

_L = 16
_W = 8
_CH = 64
_OPAD = 1


def _gather_kernel(B, D, b_per_w, NC):
    mesh = plsc.VectorSubcoreMesh(core_axis_name="c", subcore_axis_name="s")
    n_ch = b_per_w // _CH
    n_grp = _CH // _L

    @functools.partial(
        pl.kernel,
        mesh=mesh,
        out_type=jax.ShapeDtypeStruct((D, B), jnp.float32),
        compiler_params=pltpu.CompilerParams(needs_layout_passes=False),
        scratch_types=[
            pltpu.VMEM((b_per_w,), jnp.int32),
            pltpu.VMEM((_CH, _W, D), jnp.float32),
            pltpu.VMEM((D, b_per_w + _OPAD), jnp.float32),
            pltpu.SemaphoreType.DMA,
        ],
    )
    def k(idx_hbm, tab_hbm, out_hbm, idx_v, blk_v, out_v, sem):
        wid = lax.axis_index("s") * NC + lax.axis_index("c")
        base = wid * b_per_w
        pltpu.sync_copy(idx_hbm.at[pl.ds(base, b_per_w)], idx_v)
        iota = lax.iota(jnp.int32, _L)
        half = _L * (D // _L - 1)

        def ch_body(ch, _):
            off = ch * _CH
            for g in range(n_grp):
                idx16 = idx_v[pl.ds(off + g * _L, _L)]
                s16 = lax.shift_left(lax.shift_right_logical(idx16, 3), 3)
                for l in range(_L):
                    s = pl.multiple_of(s16[l], _W)
                    pltpu.async_copy(
                        tab_hbm.at[pl.ds(s, _W), :],
                        blk_v.at[g * _L + l],
                        sem,
                    )
            for g in range(n_grp):
                idx16 = idx_v[pl.ds(off + g * _L, _L)]
                c16 = idx16 & (_W - 1)
                for l in range(_L):
                    pltpu.make_async_copy(
                        tab_hbm.at[pl.ds(0, _W), :],
                        blk_v.at[g * _L + l],
                        sem,
                    ).wait()
                for l in range(_L):
                    j = g * _L + l
                    c = c16[l]
                    pos = off + j
                    lo = blk_v[j, c, pl.ds(0, _L)]
                    hi = blk_v[j, c, pl.ds(half, _L)]
                    plsc.store_scatter(out_v, [iota, iota * 0 + pos], lo)
                    plsc.store_scatter(
                        out_v, [half + iota, iota * 0 + pos], hi
                    )
            return 0

        lax.fori_loop(0, n_ch, ch_body, 0)
        pltpu.sync_copy(
            out_v.at[:, pl.ds(0, b_per_w)],
            out_hbm.at[:, pl.ds(base, b_per_w)],
        )

    return k


def kernel(node_id, node_table):
    (B,) = node_id.shape
    V, D = node_table.shape
    info = plsc.get_sparse_core_info()
    NC, NS = info.num_cores, info.num_subcores
    NW = NC * NS
    b_per_w = B // NW
    idx = node_id.astype(jnp.int32)
    out_t = _gather_kernel(B, D, b_per_w, NC)(idx, node_table)
    return out_t.T


# speedup vs baseline: 4.0200x; 2.5083x over previous
import functools

import jax
import jax.numpy as jnp
from jax import lax
from jax.experimental import pallas as pl
from jax.experimental.pallas import tpu as pltpu
from jax.experimental.pallas import tpu_sc as plsc

_L = 16
_NB = 128
_CH = 16
_PAD = 1


def _gather_kernel(B, D, V, b_per_w, NC):
    mesh = plsc.VectorSubcoreMesh(core_axis_name="c", subcore_axis_name="s")
    n_ch = b_per_w // _CH
    n_grp = _CH // _L
    half = _L * (D // _L - 1)

    @functools.partial(
        pl.kernel,
        mesh=mesh,
        out_type=jax.ShapeDtypeStruct((D, B), jnp.float32),
        compiler_params=pltpu.CompilerParams(needs_layout_passes=False),
        scratch_types=[
            pltpu.VMEM((b_per_w,), jnp.int32),
            pltpu.VMEM((_CH, D, _NB), jnp.float32),
            pltpu.VMEM((D, b_per_w), jnp.float32),
            pltpu.SemaphoreType.DMA,
        ],
    )
    def k(idx_hbm, tab_hbm, out_hbm, idx_v, blk_v, out_v, sem):
        wid = lax.axis_index("s") * NC + lax.axis_index("c")
        base = wid * b_per_w
        pltpu.sync_copy(idx_hbm.at[pl.ds(base, b_per_w)], idx_v)
        iota = lax.iota(jnp.int32, _L)

        def ch_body(ch, _):
            off = ch * _CH
            for g in range(n_grp):
                idx16 = idx_v[pl.ds(off + g * _L, _L)]
                s16 = lax.shift_left(lax.shift_right_logical(idx16, 7), 7)
                for l in range(_L):
                    s = pl.multiple_of(s16[l], _NB)
                    pltpu.async_copy(
                        tab_hbm.at[:, pl.ds(s, _NB)],
                        blk_v.at[g * _L + l],
                        sem,
                    )
            for g in range(n_grp):
                idx16 = idx_v[pl.ds(off + g * _L, _L)]
                c16 = idx16 & (_NB - 1)
                j16 = g * _L + iota
                for l in range(_L):
                    pltpu.make_async_copy(
                        tab_hbm.at[:, pl.ds(0, _NB)],
                        blk_v.at[g * _L + l],
                        sem,
                    ).wait()
                for f in range(D):
                    val = plsc.load_gather(blk_v, [j16, iota * 0 + f, c16])
                    out_v[f, pl.ds(off + g * _L, _L)] = val
            return 0

        lax.fori_loop(0, n_ch, ch_body, 0)
        pltpu.sync_copy(out_v, out_hbm.at[:, pl.ds(base, b_per_w)])

    return k


def kernel(node_id, node_table):
    (B,) = node_id.shape
    V, D = node_table.shape
    info = plsc.get_sparse_core_info()
    NC, NS = info.num_cores, info.num_subcores
    NW = NC * NS
    b_per_w = B // NW
    idx = node_id.astype(jnp.int32)
    out_t = _gather_kernel(B, D, V, b_per_w, NC)(idx, node_table.T)
    return out_t.T
